# baseline (speedup 1.0000x reference)
import jax
import jax.numpy as jnp
from jax.experimental import pallas as pl
from jax.experimental.pallas import tpu as pltpu

_N_EXPERTS = 8
_D_MODEL = 768
_D_FF = 2 * _D_MODEL
_N_TOKENS = 4096
_CAPACITY = _N_TOKENS // _N_EXPERTS
_CHUNK = 128
_N_CHUNKS = _N_TOKENS // _CHUNK
_SENTINEL = _N_EXPERTS * _CAPACITY


def _router_kernel(x_ref, rw_ref, rb_ref, g_ref, scale_ref, ps_ref):
    x = x_ref[:]
    logits = jnp.dot(x, rw_ref[:], preferred_element_type=jnp.float32)
    logits = logits + rb_ref[:]
    m = jnp.max(logits, axis=-1, keepdims=True)
    e = jnp.exp(logits - m)
    probs = e / jnp.sum(e, axis=-1, keepdims=True)
    pmax = jnp.max(probs, axis=-1, keepdims=True)
    eidx = jax.lax.broadcasted_iota(jnp.int32, (_N_TOKENS, _N_EXPERTS), 1)
    route = jnp.min(
        jnp.where(probs >= pmax, eidx, _N_EXPERTS), axis=-1, keepdims=True
    )
    onehot = (eidx == route).astype(jnp.float32)

    r_io = jax.lax.broadcasted_iota(jnp.int32, (_CHUNK, _CHUNK), 0)
    c_io = jax.lax.broadcasted_iota(jnp.int32, (_CHUNK, _CHUNK), 1)
    tril = (r_io >= c_io).astype(jnp.float32)
    carry = jnp.zeros((1, _N_EXPERTS), jnp.float32)
    for c in range(_N_CHUNKS):
        sl = slice(c * _CHUNK, (c + 1) * _CHUNK)
        oh_c = onehot[sl]
        cum = jnp.dot(tril, oh_c, preferred_element_type=jnp.float32) + carry
        carry = cum[_CHUNK - 1 : _CHUNK, :]
        pos = jnp.sum(cum * oh_c, axis=-1, keepdims=True) - 1.0
        pos_i = pos.astype(jnp.int32)
        keep = pos_i < _CAPACITY
        rc = route[sl]
        g_ref[sl] = jnp.where(keep, rc * _CAPACITY + pos_i, _SENTINEL)
        pm_c = pmax[sl]
        scale_ref[sl] = pm_c
        ps_ref[sl] = jnp.where(keep, 0.0, pm_c)


def _ffn_kernel(
    x_ref, gl_ref, gs_ref, scale_ref, ps_ref,
    w1_ref, b1_ref, w2_ref, b2_ref, out_ref,
):
    ex = pl.program_id(0)
    s_base = ex * _CAPACITY

    srow = jax.lax.broadcasted_iota(jnp.int32, (_CAPACITY, 1), 0) + s_base
    gl = gl_ref[:]
    parts = []
    for c in range(_N_CHUNKS):
        parts.append((srow == gl[c][None, :]).astype(jnp.bfloat16))
    disp = jnp.concatenate(parts, axis=1)
    buf = jnp.dot(disp, x_ref[:], preferred_element_type=jnp.float32)

    h = jnp.dot(
        buf.astype(jnp.bfloat16), w1_ref[0], preferred_element_type=jnp.float32
    )
    h = jnp.maximum(h + b1_ref[0], 0.0)
    ob = jnp.dot(
        h.astype(jnp.bfloat16), w2_ref[0], preferred_element_type=jnp.float32
    )
    ob = (ob + b2_ref[0]).astype(jnp.bfloat16)

    scol = jax.lax.broadcasted_iota(jnp.int32, (1, _CAPACITY), 1) + s_base
    dispT = (gs_ref[:] == scol).astype(jnp.bfloat16)
    comb = jnp.dot(dispT, ob, preferred_element_type=jnp.float32)

    @pl.when(ex == 0)
    def _():
        out_ref[:] = comb

    @pl.when(ex > 0)
    def _():
        out_ref[:] = out_ref[:] + comb

    @pl.when(ex == _N_EXPERTS - 1)
    def _():
        out_ref[:] = out_ref[:] * scale_ref[:] + ps_ref[:] * x_ref[:].astype(
            jnp.float32
        )


def kernel(x, router_w, router_b, w1, b1, w2, b2):
    g, scale, ps = pl.pallas_call(
        _router_kernel,
        out_shape=(
            jax.ShapeDtypeStruct((_N_TOKENS, 1), jnp.int32),
            jax.ShapeDtypeStruct((_N_TOKENS, 1), jnp.float32),
            jax.ShapeDtypeStruct((_N_TOKENS, 1), jnp.float32),
        ),
    )(x, router_w, router_b.reshape(1, _N_EXPERTS))

    gl = g.reshape(_N_CHUNKS, _CHUNK)
    x_bf = x.astype(jnp.bfloat16)
    w1_bf = w1.astype(jnp.bfloat16)
    w2_bf = w2.astype(jnp.bfloat16)

    full = lambda *shape: pl.BlockSpec(shape, lambda e: (0,) * len(shape))
    out = pl.pallas_call(
        _ffn_kernel,
        grid=(_N_EXPERTS,),
        in_specs=[
            full(_N_TOKENS, _D_MODEL),
            full(_N_CHUNKS, _CHUNK),
            full(_N_TOKENS, 1),
            full(_N_TOKENS, 1),
            full(_N_TOKENS, 1),
            pl.BlockSpec((1, _D_MODEL, _D_FF), lambda e: (e, 0, 0)),
            pl.BlockSpec((1, 1, _D_FF), lambda e: (e, 0, 0)),
            pl.BlockSpec((1, _D_FF, _D_MODEL), lambda e: (e, 0, 0)),
            pl.BlockSpec((1, 1, _D_MODEL), lambda e: (e, 0, 0)),
        ],
        out_specs=full(_N_TOKENS, _D_MODEL),
        out_shape=jax.ShapeDtypeStruct((_N_TOKENS, _D_MODEL), jnp.float32),
        compiler_params=pltpu.CompilerParams(
            dimension_semantics=("arbitrary",),
        ),
    )(
        x_bf, gl, g, scale, ps,
        w1_bf, b1.reshape(_N_EXPERTS, 1, _D_FF),
        w2_bf, b2.reshape(_N_EXPERTS, 1, _D_MODEL),
    )
    return out
